# baseline (speedup 1.0000x reference)
import jax
import jax.numpy as jnp
from jax import lax
from jax.experimental import pallas as pl
from jax.experimental.pallas import tpu as pltpu
from jax.experimental.pallas import tpu_sc as plsc

N = 10000
E = 320000
D = 128
OUT = 128

NC = 2
NS = 16
W = 128
NP = 4
DC = D // NP
EPW = E // NS
NWIN = 160
EPWP = NWIN * W
NPAD = 10112
RPW = NPAD // NS
NBUF = 4


def _spmm_body(x0_hbm, x1_hbm, x2_hbm, x3_hbm, rows_hbm, cols_hbm, vals_hbm,
               out_hbm, rows_v, cols_v, vals_v,
               gbufb0, gbufb1, gbufb2, gbufb3,
               gbuf0, gbuf1, gbuf2, gbuf3, zbuf, acc,
               gsem0, gsem1, gsem2, gsem3, ssem0, ssem1, ssem2, ssem3):
    c = lax.axis_index("c")
    s = lax.axis_index("s")

    pltpu.sync_copy(rows_hbm.at[c, s], rows_v)
    pltpu.sync_copy(cols_hbm.at[c, s], cols_v)
    pltpu.sync_copy(vals_hbm.at[c, s], vals_v)

    zero = jnp.zeros((16,), jnp.float32)
    base = s * RPW

    def zrow(i, carry):
        for j in range(DC // 16):
            zbuf[i, pl.ds(16 * j, 16)] = zero
        return carry

    lax.fori_loop(0, W, zrow, 0)

    def zero_acc_slice():
        nfull = RPW // W
        rem = RPW % W
        for k in range(nfull):
            pltpu.async_copy(zbuf, acc.at[pl.ds(base + k * W, W)], ssem0)
        if rem:
            pltpu.async_copy(zbuf.at[pl.ds(0, rem)],
                             acc.at[pl.ds(base + nfull * W, rem)], ssem0)
        for k in range(nfull):
            pltpu.make_async_copy(zbuf, acc.at[pl.ds(base + k * W, W)],
                                  ssem0).wait()
        if rem:
            pltpu.make_async_copy(zbuf.at[pl.ds(0, rem)],
                                  acc.at[pl.ds(base + nfull * W, rem)],
                                  ssem0).wait()

    zero_acc_slice()
    plsc.subcore_barrier()

    xs = (x0_hbm, x1_hbm, x2_hbm, x3_hbm)
    bufs = ((gbufb0, gbuf0, gsem0, ssem0), (gbufb1, gbuf1, gsem1, ssem1),
            (gbufb2, gbuf2, gsem2, ssem2), (gbufb3, gbuf3, gsem3, ssem3))
    for p in range(NP):
        xp = xs[p]

        def scale(gbb, gb, w):
            def sgroup(g, c2):
                vv = vals_v[w, pl.ds(g * 16, 16)]
                for l in range(16):
                    v = vv[l]
                    i = g * 16 + l
                    row = gbb[i, :]
                    ev, od = plsc.unpack(row, format=plsc.PackFormat.INTERLEAVED)
                    gb[i, pl.ds(0, 16)] = ev * v
                    gb[i, pl.ds(16, 16)] = od * v
                return c2

            lax.fori_loop(0, W // 16, sgroup, 0)

        def block(b, w, wait_prev_scatter, start_next_gather):
            gbb, gb, gs, ss = bufs[b]
            b2 = (b + 2) % NBUF
            gbb2, gb2, gs2, ss2 = bufs[b2]
            pltpu.make_async_copy(xp.at[cols_v.at[w]], gbb, gs).wait()
            scale(gbb, gb, w)
            pltpu.async_copy(gb, acc.at[rows_v.at[w]], ss, add=True)
            if wait_prev_scatter:
                pltpu.make_async_copy(gb2, acc.at[rows_v.at[w]], ss2).wait()
            if start_next_gather:
                pltpu.async_copy(xp.at[cols_v.at[w + 2]], gbb2, gs2)

        pltpu.async_copy(xp.at[cols_v.at[0]], gbufb0, gsem0)
        pltpu.async_copy(xp.at[cols_v.at[1]], gbufb1, gsem1)
        block(0, 0, False, True)
        block(1, 1, False, True)

        def qblock(g, carry):
            for b4 in range(NBUF):
                block((b4 + 2) % NBUF, 4 * g + 2 + b4, True, True)
            return carry

        lax.fori_loop(0, (NWIN - 4) // 4, qblock, 0)
        block(2, NWIN - 2, True, False)
        block(3, NWIN - 1, True, False)
        pltpu.make_async_copy(gbuf2, acc.at[rows_v.at[0]], ssem2).wait()
        pltpu.make_async_copy(gbuf3, acc.at[rows_v.at[0]], ssem3).wait()

        plsc.subcore_barrier()
        pltpu.sync_copy(acc.at[pl.ds(base, RPW)],
                        out_hbm.at[c, p, pl.ds(base, RPW)])
        if p < NP - 1:
            zero_acc_slice()
            plsc.subcore_barrier()


def _spmm_pair(xc, rows, cols, vals):
    mesh = plsc.VectorSubcoreMesh(core_axis_name="c", subcore_axis_name="s")
    f = pl.kernel(
        _spmm_body,
        out_type=jax.ShapeDtypeStruct((NC, NP, NPAD, DC), jnp.float32),
        mesh=mesh,
        scratch_types=[
            pltpu.VMEM((NWIN, W), jnp.int32),
            pltpu.VMEM((NWIN, W), jnp.int32),
            pltpu.VMEM((NWIN, W), jnp.float32),
            pltpu.VMEM((W, DC), jnp.bfloat16),
            pltpu.VMEM((W, DC), jnp.bfloat16),
            pltpu.VMEM((W, DC), jnp.bfloat16),
            pltpu.VMEM((W, DC), jnp.bfloat16),
            pltpu.VMEM((W, DC), jnp.float32),
            pltpu.VMEM((W, DC), jnp.float32),
            pltpu.VMEM((W, DC), jnp.float32),
            pltpu.VMEM((W, DC), jnp.float32),
            pltpu.VMEM((W, DC), jnp.float32),
            pltpu.VMEM_SHARED((NPAD, DC), jnp.float32),
            pltpu.SemaphoreType.DMA,
            pltpu.SemaphoreType.DMA,
            pltpu.SemaphoreType.DMA,
            pltpu.SemaphoreType.DMA,
            pltpu.SemaphoreType.DMA,
            pltpu.SemaphoreType.DMA,
            pltpu.SemaphoreType.DMA,
            pltpu.SemaphoreType.DMA,
        ],
        compiler_params=pltpu.CompilerParams(use_tc_tiling_on_sc=False,
                                             needs_layout_passes=False),
    )
    return f(xc[0], xc[1], xc[2], xc[3], rows, cols, vals)


BN_BLK = 1000


def _fc_body(o00, o01, o02, o03, o10, o11, o12, o13,
             b1_ref, b2_ref, bias_ref, y_ref, st_ref):
    o0 = (o00, o01, o02, o03)
    o1 = (o10, o11, o12, o13)
    y = jnp.broadcast_to(bias_ref[...], (BN_BLK, OUT)).astype(jnp.float32)
    for p in range(NP):
        sl = pl.ds(p * DC, DC)
        y = y + jnp.dot(o0[p][0, 0], b1_ref[sl, :],
                        preferred_element_type=jnp.float32)
        y = y + jnp.dot(o1[p][0, 0], b2_ref[sl, :],
                        preferred_element_type=jnp.float32)
    y_ref[...] = y

    @pl.when(pl.program_id(0) == 0)
    def _init():
        st_ref[...] = jnp.zeros_like(st_ref)

    upd = jnp.concatenate(
        [jnp.sum(y, axis=0, keepdims=True),
         jnp.sum(y * y, axis=0, keepdims=True),
         jnp.zeros((6, OUT), jnp.float32)], axis=0)
    st_ref[...] = st_ref[...] + upd


def _bn_body(y_ref, st_ref, g_ref, b_ref, out_ref):
    mean = st_ref[0, :] / N
    var = st_ref[1, :] / N - mean * mean
    scale = g_ref[0, :] * lax.rsqrt(var + 1e-5)
    out_ref[...] = (y_ref[...] - mean[None, :]) * scale[None, :] + b_ref[...]


_PERM = tuple(32 * p + k for p in range(NP)
              for k in list(range(0, DC, 2)) + list(range(1, DC, 2)))


def _dense_tail(o, fc_weight, fc_bias, bn_gamma, bn_beta):
    perm = jnp.asarray(_PERM, dtype=jnp.int32)
    b1 = fc_weight[:, :D].T[perm, :]
    b2 = fc_weight[:, D:].T[perm, :]
    bias = fc_bias[None, :]
    nblk = N // BN_BLK
    ospecs = [pl.BlockSpec((1, 1, BN_BLK, DC),
                           lambda i, m=m, p=p: (m, p, i, 0))
              for m in range(NC) for p in range(NP)]
    y, st = pl.pallas_call(
        _fc_body,
        grid=(nblk,),
        in_specs=ospecs + [
            pl.BlockSpec((D, OUT), lambda i: (0, 0)),
            pl.BlockSpec((D, OUT), lambda i: (0, 0)),
            pl.BlockSpec((1, OUT), lambda i: (0, 0)),
        ],
        out_specs=[
            pl.BlockSpec((BN_BLK, OUT), lambda i: (i, 0)),
            pl.BlockSpec((8, OUT), lambda i: (0, 0)),
        ],
        out_shape=[
            jax.ShapeDtypeStruct((N, OUT), jnp.float32),
            jax.ShapeDtypeStruct((8, OUT), jnp.float32),
        ],
    )(o, o, o, o, o, o, o, o, b1, b2, bias)
    out = pl.pallas_call(
        _bn_body,
        grid=(nblk,),
        in_specs=[
            pl.BlockSpec((BN_BLK, OUT), lambda i: (i, 0)),
            pl.BlockSpec((8, OUT), lambda i: (0, 0)),
            pl.BlockSpec((1, OUT), lambda i: (0, 0)),
            pl.BlockSpec((1, OUT), lambda i: (0, 0)),
        ],
        out_specs=pl.BlockSpec((BN_BLK, OUT), lambda i: (i, 0)),
        out_shape=jax.ShapeDtypeStruct((N, OUT), jnp.float32),
    )(y, st, bn_gamma[None, :], bn_beta[None, :])
    return out


def _pad_edges(a, pad_vec):
    a = a.reshape(NS, EPW)
    pad = jnp.broadcast_to(pad_vec[None, :], (NS, EPWP - EPW))
    return jnp.concatenate([a, pad], axis=1)


def kernel(x, W1_indices, W1_values, W2_indices, W2_values,
           fc_weight, fc_bias, bn_gamma, bn_beta):
    xc = x.reshape(N, NP, DC).transpose(1, 0, 2).astype(jnp.bfloat16)
    npad_e = EPWP - EPW
    pad_cols = (jnp.arange(npad_e, dtype=jnp.int32) * 37) % N
    pad_rows = (jnp.arange(npad_e, dtype=jnp.int32) * 13) % NPAD
    pad_vals = jnp.zeros((npad_e,), jnp.float32)
    rows = jnp.stack([_pad_edges(W1_indices[0], pad_rows),
                      _pad_edges(W2_indices[0], pad_rows)])
    cols = jnp.stack([_pad_edges(W1_indices[1], pad_cols),
                      _pad_edges(W2_indices[1], pad_cols)])
    vals = jnp.stack([_pad_edges(W1_values, pad_vals),
                      _pad_edges(W2_values, pad_vals)])
    rows = rows.reshape(NC, NS, NWIN, W)
    cols = cols.reshape(NC, NS, NWIN, W)
    vals = vals.reshape(NC, NS, NWIN, W)
    o = _spmm_pair(xc, rows, cols, vals)
    return _dense_tail(o, fc_weight, fc_bias, bn_gamma, bn_beta)


# speedup vs baseline: 1.3216x; 1.3216x over previous
import jax
import jax.numpy as jnp
from jax import lax
from jax.experimental import pallas as pl
from jax.experimental.pallas import tpu as pltpu
from jax.experimental.pallas import tpu_sc as plsc

N = 10000
E = 320000
D = 128
OUT = 128

NC = 2
NS = 16
W = 128
NP = 4
DC = D // NP
EPW = E // NS
NWIN = 160
EPWP = NWIN * W
NPAD = 10112
RPW = NPAD // NS
NBUF = 4


def _spmm_body(x0_hbm, x1_hbm, x2_hbm, x3_hbm, rows_hbm, cols_hbm, vals_hbm,
               out_hbm, rows_v, cols_v, vals_v,
               gbuf0, gbuf1, gbuf2, gbuf3, zbuf, acc,
               gsem0, gsem1, gsem2, gsem3, ssem0, ssem1, ssem2, ssem3):
    c = lax.axis_index("c")
    s = lax.axis_index("s")

    pltpu.sync_copy(rows_hbm.at[c, s], rows_v)
    pltpu.sync_copy(cols_hbm.at[c, s], cols_v)
    pltpu.sync_copy(vals_hbm.at[c, s], vals_v)

    zero = jnp.zeros((16,), jnp.float32)
    base = s * RPW

    def zrow(i, carry):
        for j in range(DC // 16):
            zbuf[i, pl.ds(16 * j, 16)] = zero
        return carry

    lax.fori_loop(0, W, zrow, 0)

    def zero_acc_slice():
        nfull = RPW // W
        rem = RPW % W
        for k in range(nfull):
            pltpu.async_copy(zbuf, acc.at[pl.ds(base + k * W, W)], ssem0)
        if rem:
            pltpu.async_copy(zbuf.at[pl.ds(0, rem)],
                             acc.at[pl.ds(base + nfull * W, rem)], ssem0)
        for k in range(nfull):
            pltpu.make_async_copy(zbuf, acc.at[pl.ds(base + k * W, W)],
                                  ssem0).wait()
        if rem:
            pltpu.make_async_copy(zbuf.at[pl.ds(0, rem)],
                                  acc.at[pl.ds(base + nfull * W, rem)],
                                  ssem0).wait()

    zero_acc_slice()
    plsc.subcore_barrier()

    xs = (x0_hbm, x1_hbm, x2_hbm, x3_hbm)
    bufs = ((gbuf0, gsem0, ssem0), (gbuf1, gsem1, ssem1),
            (gbuf2, gsem2, ssem2), (gbuf3, gsem3, ssem3))
    for p in range(NP):
        xp = xs[p]

        def scale(gb, w):
            def sgroup(g, c2):
                vv = vals_v[w, pl.ds(g * 16, 16)]
                for l in range(16):
                    v = vv[l]
                    i = g * 16 + l
                    for j in range(DC // 16):
                        sl = pl.ds(16 * j, 16)
                        gb[i, sl] = gb[i, sl] * v
                return c2

            lax.fori_loop(0, W // 16, sgroup, 0)

        def block(b, w, wait_prev_scatter, start_next_gather):
            gb, gs, ss = bufs[b]
            b2 = (b + 2) % NBUF
            gb2, gs2, ss2 = bufs[b2]
            pltpu.make_async_copy(xp.at[cols_v.at[w]], gb, gs).wait()
            scale(gb, w)
            pltpu.async_copy(gb, acc.at[rows_v.at[w]], ss, add=True)
            if wait_prev_scatter:
                pltpu.make_async_copy(gb2, acc.at[rows_v.at[w]], ss2).wait()
            if start_next_gather:
                pltpu.async_copy(xp.at[cols_v.at[w + 2]], gb2, gs2)

        pltpu.async_copy(xp.at[cols_v.at[0]], gbuf0, gsem0)
        pltpu.async_copy(xp.at[cols_v.at[1]], gbuf1, gsem1)
        block(0, 0, False, True)
        block(1, 1, False, True)

        def qblock(g, carry):
            for b4 in range(NBUF):
                block((b4 + 2) % NBUF, 4 * g + 2 + b4, True, True)
            return carry

        lax.fori_loop(0, (NWIN - 4) // 4, qblock, 0)
        block(2, NWIN - 2, True, False)
        block(3, NWIN - 1, True, False)
        pltpu.make_async_copy(gbuf2, acc.at[rows_v.at[0]], ssem2).wait()
        pltpu.make_async_copy(gbuf3, acc.at[rows_v.at[0]], ssem3).wait()

        plsc.subcore_barrier()
        pltpu.sync_copy(acc.at[pl.ds(base, RPW)],
                        out_hbm.at[c, p, pl.ds(base, RPW)])
        if p < NP - 1:
            zero_acc_slice()
            plsc.subcore_barrier()


def _spmm_pair(xc, rows, cols, vals):
    mesh = plsc.VectorSubcoreMesh(core_axis_name="c", subcore_axis_name="s")
    f = pl.kernel(
        _spmm_body,
        out_type=jax.ShapeDtypeStruct((NC, NP, NPAD, DC), jnp.float32),
        mesh=mesh,
        scratch_types=[
            pltpu.VMEM((NWIN, W), jnp.int32),
            pltpu.VMEM((NWIN, W), jnp.int32),
            pltpu.VMEM((NWIN, W), jnp.float32),
            pltpu.VMEM((W, DC), jnp.float32),
            pltpu.VMEM((W, DC), jnp.float32),
            pltpu.VMEM((W, DC), jnp.float32),
            pltpu.VMEM((W, DC), jnp.float32),
            pltpu.VMEM((W, DC), jnp.float32),
            pltpu.VMEM_SHARED((NPAD, DC), jnp.float32),
            pltpu.SemaphoreType.DMA,
            pltpu.SemaphoreType.DMA,
            pltpu.SemaphoreType.DMA,
            pltpu.SemaphoreType.DMA,
            pltpu.SemaphoreType.DMA,
            pltpu.SemaphoreType.DMA,
            pltpu.SemaphoreType.DMA,
            pltpu.SemaphoreType.DMA,
        ],
        compiler_params=pltpu.CompilerParams(use_tc_tiling_on_sc=False),
    )
    return f(xc[0], xc[1], xc[2], xc[3], rows, cols, vals)


BN_BLK = 1000


def _fc_body(o00, o01, o02, o03, o10, o11, o12, o13,
             b1_ref, b2_ref, bias_ref, y_ref, st_ref):
    o0 = (o00, o01, o02, o03)
    o1 = (o10, o11, o12, o13)
    y = jnp.broadcast_to(bias_ref[...], (BN_BLK, OUT)).astype(jnp.float32)
    for p in range(NP):
        sl = pl.ds(p * DC, DC)
        y = y + jnp.dot(o0[p][0, 0], b1_ref[sl, :],
                        preferred_element_type=jnp.float32)
        y = y + jnp.dot(o1[p][0, 0], b2_ref[sl, :],
                        preferred_element_type=jnp.float32)
    y_ref[...] = y

    @pl.when(pl.program_id(0) == 0)
    def _init():
        st_ref[...] = jnp.zeros_like(st_ref)

    upd = jnp.concatenate(
        [jnp.sum(y, axis=0, keepdims=True),
         jnp.sum(y * y, axis=0, keepdims=True),
         jnp.zeros((6, OUT), jnp.float32)], axis=0)
    st_ref[...] = st_ref[...] + upd


def _bn_body(y_ref, st_ref, g_ref, b_ref, out_ref):
    mean = st_ref[0, :] / N
    var = st_ref[1, :] / N - mean * mean
    scale = g_ref[0, :] * lax.rsqrt(var + 1e-5)
    out_ref[...] = (y_ref[...] - mean[None, :]) * scale[None, :] + b_ref[...]


def _dense_tail(o, fc_weight, fc_bias, bn_gamma, bn_beta):
    b1 = fc_weight[:, :D].T
    b2 = fc_weight[:, D:].T
    bias = fc_bias[None, :]
    nblk = N // BN_BLK
    ospecs = [pl.BlockSpec((1, 1, BN_BLK, DC),
                           lambda i, m=m, p=p: (m, p, i, 0))
              for m in range(NC) for p in range(NP)]
    y, st = pl.pallas_call(
        _fc_body,
        grid=(nblk,),
        in_specs=ospecs + [
            pl.BlockSpec((D, OUT), lambda i: (0, 0)),
            pl.BlockSpec((D, OUT), lambda i: (0, 0)),
            pl.BlockSpec((1, OUT), lambda i: (0, 0)),
        ],
        out_specs=[
            pl.BlockSpec((BN_BLK, OUT), lambda i: (i, 0)),
            pl.BlockSpec((8, OUT), lambda i: (0, 0)),
        ],
        out_shape=[
            jax.ShapeDtypeStruct((N, OUT), jnp.float32),
            jax.ShapeDtypeStruct((8, OUT), jnp.float32),
        ],
    )(o, o, o, o, o, o, o, o, b1, b2, bias)
    out = pl.pallas_call(
        _bn_body,
        grid=(nblk,),
        in_specs=[
            pl.BlockSpec((BN_BLK, OUT), lambda i: (i, 0)),
            pl.BlockSpec((8, OUT), lambda i: (0, 0)),
            pl.BlockSpec((1, OUT), lambda i: (0, 0)),
            pl.BlockSpec((1, OUT), lambda i: (0, 0)),
        ],
        out_specs=pl.BlockSpec((BN_BLK, OUT), lambda i: (i, 0)),
        out_shape=jax.ShapeDtypeStruct((N, OUT), jnp.float32),
    )(y, st, bn_gamma[None, :], bn_beta[None, :])
    return out


def _pad_edges(a, pad_vec):
    a = a.reshape(NS, EPW)
    pad = jnp.broadcast_to(pad_vec[None, :], (NS, EPWP - EPW))
    return jnp.concatenate([a, pad], axis=1)


def kernel(x, W1_indices, W1_values, W2_indices, W2_values,
           fc_weight, fc_bias, bn_gamma, bn_beta):
    xc = x.reshape(N, NP, DC).transpose(1, 0, 2)
    npad_e = EPWP - EPW
    pad_cols = (jnp.arange(npad_e, dtype=jnp.int32) * 37) % N
    pad_rows = (jnp.arange(npad_e, dtype=jnp.int32) * 13) % NPAD
    pad_vals = jnp.zeros((npad_e,), jnp.float32)
    rows = jnp.stack([_pad_edges(W1_indices[0], pad_rows),
                      _pad_edges(W2_indices[0], pad_rows)])
    cols = jnp.stack([_pad_edges(W1_indices[1], pad_cols),
                      _pad_edges(W2_indices[1], pad_cols)])
    vals = jnp.stack([_pad_edges(W1_values, pad_vals),
                      _pad_edges(W2_values, pad_vals)])
    rows = rows.reshape(NC, NS, NWIN, W)
    cols = cols.reshape(NC, NS, NWIN, W)
    vals = vals.reshape(NC, NS, NWIN, W)
    o = _spmm_pair(xc, rows, cols, vals)
    return _dense_tail(o, fc_weight, fc_bias, bn_gamma, bn_beta)
